# baseline (speedup 1.0000x reference)
import functools

import jax
import jax.numpy as jnp
from jax import lax
from jax.experimental import pallas as pl
from jax.experimental.pallas import tpu as pltpu
from jax.experimental.pallas import tpu_sc as plsc

VOCAB = 1000000
EMB = 64
BATCH = 4096
SEQ = 200
SEQP = 256

NW = 32
ROWS_W = BATCH // NW
LANES = 16
NVREG = 13
GROWS = NVREG * LANES
RB = 32
NIB = ROWS_W // RB

_mesh = plsc.VectorSubcoreMesh(core_axis_name="c", subcore_axis_name="s")


@functools.partial(
    pl.kernel,
    mesh=_mesh,
    out_type=jax.ShapeDtypeStruct((BATCH, SEQ // 2, 2 * EMB), jnp.float32),
    scratch_types=[
        pltpu.VMEM((2, RB, SEQP), jnp.int32),
        pltpu.VMEM((2, GROWS, 2 * EMB), jnp.float32),
        pltpu.VMEM((2, SEQ // 2, 2 * EMB), jnp.float32),
        pltpu.SemaphoreType.DMA,
        pltpu.SemaphoreType.DMA,
        pltpu.SemaphoreType.DMA,
        pltpu.SemaphoreType.DMA,
        pltpu.SemaphoreType.DMA,
        pltpu.SemaphoreType.DMA,
    ],
    compiler_params=pltpu.CompilerParams(use_tc_tiling_on_sc=True),
)
def _gather_kernel(x_hbm, table_hbm, out_hbm, idx_v, pair_v, rows_v,
                   isem0, isem1, gsem0, gsem1, wsem0, wsem1):
    isems = (isem0, isem1)
    gsems = (gsem0, gsem1)
    wsems = (wsem0, wsem1)
    wid = lax.axis_index("s") * 2 + lax.axis_index("c")
    base = wid * ROWS_W

    def stage_idx(blk, b):
        pltpu.async_copy(x_hbm.at[pl.ds(base + blk * RB, RB)],
                         idx_v.at[b], isems[b])

    def wait_idx(b):
        pltpu.make_async_copy(x_hbm.at[pl.ds(0, RB)], idx_v.at[b],
                              isems[b]).wait()

    def fire(r, ib, gb):
        rr = r % RB
        for t in range(NVREG):
            iv = idx_v[ib, rr, pl.ds(t * LANES, LANES)] >> 1
            pltpu.async_copy(table_hbm.at[iv],
                             pair_v.at[gb, pl.ds(t * LANES, LANES)],
                             gsems[gb])

    def drain_gather(gb):
        pltpu.make_async_copy(table_hbm.at[pl.ds(0, GROWS)], pair_v.at[gb],
                              gsems[gb]).wait()

    def select(r, ib, gb, wb):
        rr = r % RB

        def vgroup(g, carry):
            iv = idx_v[ib, rr, pl.ds(g * LANES, LANES)]
            hv = (iv & 1) << 6
            for k in range(LANES):
                off = hv[k]
                half = k % 2
                for c in range(4):
                    rows_v[wb, g * 8 + k // 2,
                           pl.ds(half * EMB + c * 16, 16)] = (
                        pair_v[gb, g * LANES + k, pl.ds(off + c * 16, 16)])
            return carry

        lax.fori_loop(0, NVREG - 1, vgroup, 0, unroll=False)
        iv = idx_v[ib, rr, pl.ds((NVREG - 1) * LANES, LANES)]
        hv = (iv & 1) << 6
        for k in range(SEQ - (NVREG - 1) * LANES):
            off = hv[k]
            half = k % 2
            p = (NVREG - 1) * LANES + k
            for c in range(4):
                rows_v[wb, p // 2, pl.ds(half * EMB + c * 16, 16)] = (
                    pair_v[gb, p, pl.ds(off + c * 16, 16)])

    def start_write(r, wb):
        pltpu.async_copy(rows_v.at[wb], out_hbm.at[base + r], wsems[wb])

    def wait_write(wb):
        pltpu.make_async_copy(rows_v.at[wb], out_hbm.at[0], wsems[wb]).wait()

    stage_idx(0, 0)
    stage_idx(1, 1)
    wait_idx(0)
    fire(0, 0, 0)

    def row_pair(p_, carry):
        for i in range(2):
            r = 2 * p_ + i
            gb = i
            wb = i
            ib = (r // RB) % 2
            nblk = r // RB + 1
            nxt_ib = ((r + 1) // RB) % 2

            stage_now = jnp.logical_and(
                r % RB == 0, jnp.logical_and(r > 0, nblk < NIB))
            for sb in range(2):
                @pl.when(jnp.logical_and(stage_now, nblk % 2 == sb))
                def _(sb=sb):
                    stage_idx(nblk, sb)

            @pl.when(r + 1 < ROWS_W)
            def _():
                for sb in range(2):
                    @pl.when(jnp.logical_and((r + 1) % RB == 0,
                                             nxt_ib == sb))
                    def _(sb=sb):
                        wait_idx(sb)
                fire(r + 1, nxt_ib, 1 - gb)

            drain_gather(gb)

            @pl.when(r >= 2)
            def _():
                wait_write(wb)

            select(r, ib, gb, wb)
            start_write(r, wb)
        return carry

    lax.fori_loop(0, ROWS_W // 2, row_pair, 0, unroll=False)
    wait_write(0)
    wait_write(1)


def kernel(x, table):
    xp = jnp.pad(x.astype(jnp.int32), ((0, 0), (0, SEQP - SEQ)))
    tab2 = table.reshape(VOCAB // 2, 2 * EMB)
    out2 = _gather_kernel(xp, tab2)
    return out2.reshape(BATCH, SEQ, EMB)


# speedup vs baseline: 1.0001x; 1.0001x over previous
import functools

import jax
import jax.numpy as jnp
from jax import lax
from jax.experimental import pallas as pl
from jax.experimental.pallas import tpu as pltpu
from jax.experimental.pallas import tpu_sc as plsc

VOCAB = 1000000
EMB = 64
BATCH = 4096
SEQ = 200
SEQP = 256

NW = 32
ROWS_W = BATCH // NW
LANES = 16
NVREG = 13
GROWS = NVREG * LANES
RB = 32
NIB = ROWS_W // RB

_mesh = plsc.VectorSubcoreMesh(core_axis_name="c", subcore_axis_name="s")


@functools.partial(
    pl.kernel,
    mesh=_mesh,
    out_type=jax.ShapeDtypeStruct((BATCH, SEQ // 2, 2 * EMB), jnp.float32),
    scratch_types=[
        pltpu.VMEM((2, RB, SEQP), jnp.int32),
        pltpu.VMEM((2, GROWS, 2 * EMB), jnp.float32),
        pltpu.VMEM((2, SEQ // 2, 2 * EMB), jnp.float32),
        pltpu.SemaphoreType.DMA,
        pltpu.SemaphoreType.DMA,
        pltpu.SemaphoreType.DMA,
        pltpu.SemaphoreType.DMA,
        pltpu.SemaphoreType.DMA,
        pltpu.SemaphoreType.DMA,
    ],
    compiler_params=pltpu.CompilerParams(use_tc_tiling_on_sc=True),
)
def _gather_kernel(x_hbm, table_hbm, out_hbm, idx_v, pair_v, rows_v,
                   isem0, isem1, gsem0, gsem1, wsem0, wsem1):
    isems = (isem0, isem1)
    gsems = (gsem0, gsem1)
    wsems = (wsem0, wsem1)
    wid = lax.axis_index("s") * 2 + lax.axis_index("c")
    base = wid * ROWS_W

    def stage_idx(blk, b):
        pltpu.async_copy(x_hbm.at[pl.ds(base + blk * RB, RB)],
                         idx_v.at[b], isems[b])

    def wait_idx(b):
        pltpu.make_async_copy(x_hbm.at[pl.ds(0, RB)], idx_v.at[b],
                              isems[b]).wait()

    def fire(r, ib, gb):
        rr = r % RB
        for t in range(NVREG):
            iv = idx_v[ib, rr, pl.ds(t * LANES, LANES)] >> 1
            pltpu.async_copy(table_hbm.at[iv],
                             pair_v.at[gb, pl.ds(t * LANES, LANES)],
                             gsems[gb])

    def drain_gather(gb):
        pltpu.make_async_copy(table_hbm.at[pl.ds(0, GROWS)], pair_v.at[gb],
                              gsems[gb]).wait()

    def select(r, ib, gb, wb):
        rr = r % RB

        def vgroup(g, carry):
            iv = idx_v[ib, rr, pl.ds(g * LANES, LANES)]
            bits = iv & 1
            for k in range(LANES):
                b_k = bits[k]
                half = k % 2
                row = g * LANES + k
                for c in range(4):
                    lo = pair_v[gb, row, pl.ds(c * 16, 16)]
                    hi = pair_v[gb, row, pl.ds(EMB + c * 16, 16)]
                    rows_v[wb, g * 8 + k // 2,
                           pl.ds(half * EMB + c * 16, 16)] = (
                        jnp.where(b_k != 0, hi, lo))
            return carry

        lax.fori_loop(0, NVREG - 1, vgroup, 0, unroll=False)
        iv = idx_v[ib, rr, pl.ds((NVREG - 1) * LANES, LANES)]
        bits = iv & 1
        for k in range(SEQ - (NVREG - 1) * LANES):
            b_k = bits[k]
            half = k % 2
            p = (NVREG - 1) * LANES + k
            for c in range(4):
                lo = pair_v[gb, p, pl.ds(c * 16, 16)]
                hi = pair_v[gb, p, pl.ds(EMB + c * 16, 16)]
                rows_v[wb, p // 2, pl.ds(half * EMB + c * 16, 16)] = (
                    jnp.where(b_k != 0, hi, lo))

    def start_write(r, wb):
        pltpu.async_copy(rows_v.at[wb], out_hbm.at[base + r], wsems[wb])

    def wait_write(wb):
        pltpu.make_async_copy(rows_v.at[wb], out_hbm.at[0], wsems[wb]).wait()

    stage_idx(0, 0)
    stage_idx(1, 1)
    wait_idx(0)
    fire(0, 0, 0)

    def row_pair(p_, carry):
        for i in range(2):
            r = 2 * p_ + i
            gb = i
            wb = i
            ib = (r // RB) % 2
            nblk = r // RB + 1
            nxt_ib = ((r + 1) // RB) % 2

            stage_now = jnp.logical_and(
                r % RB == 0, jnp.logical_and(r > 0, nblk < NIB))
            for sb in range(2):
                @pl.when(jnp.logical_and(stage_now, nblk % 2 == sb))
                def _(sb=sb):
                    stage_idx(nblk, sb)

            @pl.when(r + 1 < ROWS_W)
            def _():
                for sb in range(2):
                    @pl.when(jnp.logical_and((r + 1) % RB == 0,
                                             nxt_ib == sb))
                    def _(sb=sb):
                        wait_idx(sb)
                fire(r + 1, nxt_ib, 1 - gb)

            drain_gather(gb)

            @pl.when(r >= 2)
            def _():
                wait_write(wb)

            select(r, ib, gb, wb)
            start_write(r, wb)
        return carry

    lax.fori_loop(0, ROWS_W // 2, row_pair, 0, unroll=False)
    wait_write(0)
    wait_write(1)


def kernel(x, table):
    xp = jnp.pad(x.astype(jnp.int32), ((0, 0), (0, SEQP - SEQ)))
    tab2 = table.reshape(VOCAB // 2, 2 * EMB)
    out2 = _gather_kernel(xp, tab2)
    return out2.reshape(BATCH, SEQ, EMB)
